# baseline (speedup 1.0000x reference)
import functools

import jax
import jax.numpy as jnp
from jax import lax
from jax.experimental import pallas as pl
from jax.experimental.pallas import tpu as pltpu
from jax.experimental.pallas import tpu_sc as plsc

N_NODES = 10000
N_EDGES = 320000
NW = 32
EPAD = 256
NEG = -1e30


def _mm_body(x_ref, wl_ref, bl_ref, wr_ref, br_ref, ol_ref, or_ref):
    x = x_ref[...]
    ol_ref[...] = (
        jnp.dot(x, wl_ref[...], preferred_element_type=jnp.float32) + bl_ref[...]
    )
    or_ref[...] = (
        jnp.dot(x, wr_ref[...], preferred_element_type=jnp.float32) + br_ref[...]
    )


def _linear_pair(x, wl, bl, wr, br, block_rows=1000):
    n, din = x.shape
    dout = wl.shape[1]
    out_sds = jax.ShapeDtypeStruct((n, dout), jnp.float32)
    return pl.pallas_call(
        _mm_body,
        grid=(n // block_rows,),
        in_specs=[
            pl.BlockSpec((block_rows, din), lambda i: (i, 0)),
            pl.BlockSpec((din, dout), lambda i: (0, 0)),
            pl.BlockSpec((1, dout), lambda i: (0, 0)),
            pl.BlockSpec((din, dout), lambda i: (0, 0)),
            pl.BlockSpec((1, dout), lambda i: (0, 0)),
        ],
        out_specs=[
            pl.BlockSpec((block_rows, dout), lambda i: (i, 0)),
            pl.BlockSpec((block_rows, dout), lambda i: (i, 0)),
        ],
        out_shape=[out_sds, out_sds],
    )(x, wl, bl.reshape(1, -1), wr, br.reshape(1, -1))


def _sread(ref, i):
    return ref[pl.ds(i, 16)][0]


def _allsum(v, lanes):
    for s in (8, 4, 2, 1):
        v = v + v.at[lanes ^ s].get(mode="promise_in_bounds")
    return v


@functools.lru_cache(maxsize=None)
def _make_edge_kernel(D, relu):
    C = max(16, min(128, (49152 // D) // 16 * 16))
    JD = D // 16
    U = 8 if JD % 8 == 0 else (4 if JD % 4 == 0 else 1)
    mesh = plsc.VectorSubcoreMesh(core_axis_name="c", subcore_axis_name="s")

    def body(xl_h, xr_h, att_h, bias_h, src_h, dst_h, eoff_h, out_h,
             srcbuf, dstbuf, xjb, xib, attv, biasv, acc,
             orow, eoffv, sem_j, sem_i):
        lanes = lax.iota(jnp.int32, 16)
        wid = lax.axis_index("s") * 2 + lax.axis_index("c")
        pltpu.sync_copy(eoff_h, eoffv)
        pltpu.sync_copy(att_h, attv)
        pltpu.sync_copy(bias_h, biasv)
        e0 = _sread(eoffv, wid)
        e1 = _sread(eoffv, wid + 1)
        for j in range(JD):
            acc[pl.ds(16 * j, 16)] = jnp.zeros((16,), jnp.float32)
        g0 = (e0 // 8) * 8
        nchunks = jnp.where(e1 > e0, (e1 - g0 + C - 1) // C, 0)

        def fin_effects(cur_dst, den_v):
            rcp = 1.0 / den_v
            for j in range(JD):
                o_v = acc[pl.ds(16 * j, 16)] * rcp + biasv[pl.ds(16 * j, 16)]
                if relu:
                    o_v = jnp.maximum(o_v, 0.0)
                orow[0, pl.ds(16 * j, 16)] = o_v
            pltpu.sync_copy(orow, out_h.at[pl.ds(cur_dst, 1)])

        def chunk_body(c, st):
            g = pl.multiple_of(g0 + c * C, 8)
            start = jnp.maximum(e0 - g, 0)
            end = jnp.minimum(e1 - g, C)
            pltpu.sync_copy(src_h.at[pl.ds(g, C)], srcbuf)
            pltpu.sync_copy(dst_h.at[pl.ds(g, C + 16)], dstbuf)
            cj = pltpu.async_copy(xl_h.at[srcbuf], xjb, sem_j)
            ci = pltpu.async_copy(xr_h.at[dstbuf.at[pl.ds(0, C)]], xib, sem_i)
            cj.wait()
            ci.wait()

            def edge(e, s2):
                cur_dst, m, den_v = s2
                dcur = _sread(dstbuf, e)
                zero16 = jnp.zeros((16,), jnp.float32)
                NP = 4 if U >= 4 else 1

                def sblk(b, parts):
                    parts = list(parts)
                    base = b * (16 * U)
                    for t in range(U):
                        off = base + 16 * t
                        xj_v = xjb[e, pl.ds(off, 16)]
                        xi_v = xib[e, pl.ds(off, 16)]
                        z = xi_v + xj_v
                        z = jnp.maximum(z, 0.2 * z)
                        parts[t % NP] = parts[t % NP] + z * attv[pl.ds(off, 16)]
                    return tuple(parts)

                parts = lax.fori_loop(0, JD // U, sblk,
                                      tuple(zero16 for _ in range(NP)))
                sv = parts[0]
                for t in range(1, NP):
                    sv = sv + parts[t]
                s = _allsum(sv, lanes)[0]
                fin = (dcur != cur_dst) & (cur_dst >= 0)

                @pl.when(fin)
                def _():
                    fin_effects(cur_dst, den_v)

                fresh = dcur != cur_dst
                m = jnp.where(fresh, NEG, m)
                den_v = jnp.where(fresh, jnp.zeros((16,), jnp.float32), den_v)
                mn = jnp.maximum(m, s)
                r_v = jnp.exp(jnp.full((16,), m - mn, jnp.float32))
                w_v = jnp.exp(jnp.full((16,), s - mn, jnp.float32))
                den_v = den_v * r_v + w_v

                @pl.when(mn > m)
                def _():
                    def rblk(b, carry):
                        base = b * (16 * U)
                        for t in range(U):
                            off = base + 16 * t
                            acc[pl.ds(off, 16)] = acc[pl.ds(off, 16)] * r_v
                        return carry

                    lax.fori_loop(0, JD // U, rblk, 0)

                def ablk(b, carry):
                    base = b * (16 * U)
                    for t in range(U):
                        off = base + 16 * t
                        acc[pl.ds(off, 16)] = (
                            acc[pl.ds(off, 16)]
                            + w_v * xjb[e, pl.ds(off, 16)])
                    return carry

                lax.fori_loop(0, JD // U, ablk, 0)
                return (dcur, mn, den_v)

            return lax.fori_loop(start, end, edge, st)

        init = (jnp.int32(-1), jnp.float32(NEG),
                jnp.zeros((16,), jnp.float32))
        cur_dst, m, den_v = lax.fori_loop(0, nchunks, chunk_body, init)

        @pl.when(cur_dst >= 0)
        def _():
            fin_effects(cur_dst, den_v)

    return pl.kernel(
        body,
        mesh=mesh,
        compiler_params=pltpu.CompilerParams(use_tc_tiling_on_sc=False),
        out_type=jax.ShapeDtypeStruct((N_NODES, D), jnp.float32),
        scratch_types=[
            pltpu.VMEM((C,), jnp.int32),
            pltpu.VMEM((C + 16,), jnp.int32),
            pltpu.VMEM((C, D), jnp.float32),
            pltpu.VMEM((C, D), jnp.float32),
            pltpu.VMEM((D,), jnp.float32),
            pltpu.VMEM((D,), jnp.float32),
            pltpu.VMEM((D,), jnp.float32),
            pltpu.VMEM((1, D), jnp.float32),
            pltpu.VMEM((64,), jnp.int32),
            pltpu.SemaphoreType.DMA,
            pltpu.SemaphoreType.DMA,
        ],
    )


def _pad_cols(w, d):
    dout, din = w.shape
    wt = w.T
    if d > dout:
        wt = jnp.pad(wt, ((0, 0), (0, d - dout)))
    return wt


def _pad_vec(v, d):
    if d > v.shape[0]:
        v = jnp.pad(v, (0, d - v.shape[0]))
    return v


def _layer(x, srcs, dsts, eoff, p, relu):
    dout = p['Wl'].shape[0]
    D = max(16, -(-dout // 16) * 16)
    wl = _pad_cols(p['Wl'], D)
    wr = _pad_cols(p['Wr'], D)
    bl = _pad_vec(p['bl'], D)
    br = _pad_vec(p['br'], D)
    att = _pad_vec(p['att'], D)
    bias = _pad_vec(p['bias'], D)
    xl, xr = _linear_pair(x, wl, bl, wr, br)
    edge_fn = _make_edge_kernel(D, relu)
    out = edge_fn(xl, xr, att, bias, srcs, dsts, eoff)
    return out[:N_NODES]


def kernel(x, edge_index, params):
    n = x.shape[0]
    ei = edge_index.astype(jnp.int32)
    loop = jnp.arange(n, dtype=jnp.int32)
    src = jnp.concatenate([ei[0], loop])
    dst = jnp.concatenate([ei[1], loop])
    key = jnp.sort(dst * 16384 + src)
    dsts = key >> 14
    srcs = key & 16383
    bounds = jnp.array([w * n // NW for w in range(NW + 1)], jnp.int32)
    eoff = jnp.searchsorted(dsts, bounds).astype(jnp.int32)
    eoff = jnp.concatenate([eoff, jnp.zeros((64 - NW - 1,), jnp.int32)])
    srcs = jnp.concatenate([srcs, jnp.zeros((EPAD,), jnp.int32)])
    dsts = jnp.concatenate([dsts, jnp.zeros((EPAD,), jnp.int32)])

    out = x
    for i in range(4):
        out = _layer(out, srcs, dsts, eoff, params[i], True)
    coords = _layer(out, srcs, dsts, eoff, params[4], False)
    return (out, coords[:, :3])


# speedup vs baseline: 1.3895x; 1.3895x over previous
import functools

import jax
import jax.numpy as jnp
from jax import lax
from jax.experimental import pallas as pl
from jax.experimental.pallas import tpu as pltpu
from jax.experimental.pallas import tpu_sc as plsc

N_NODES = 10000
N_EDGES = 320000
NW = 32
EPAD = 256
NEG = -1e30


def _mm_body(x_ref, wl_ref, bl_ref, wr_ref, br_ref, ol_ref, or_ref):
    x = x_ref[...]
    ol_ref[...] = (
        jnp.dot(x, wl_ref[...], preferred_element_type=jnp.float32) + bl_ref[...]
    )
    or_ref[...] = (
        jnp.dot(x, wr_ref[...], preferred_element_type=jnp.float32) + br_ref[...]
    )


def _linear_pair(x, wl, bl, wr, br, block_rows=1000):
    n, din = x.shape
    dout = wl.shape[1]
    out_sds = jax.ShapeDtypeStruct((n, dout), jnp.float32)
    return pl.pallas_call(
        _mm_body,
        grid=(n // block_rows,),
        in_specs=[
            pl.BlockSpec((block_rows, din), lambda i: (i, 0)),
            pl.BlockSpec((din, dout), lambda i: (0, 0)),
            pl.BlockSpec((1, dout), lambda i: (0, 0)),
            pl.BlockSpec((din, dout), lambda i: (0, 0)),
            pl.BlockSpec((1, dout), lambda i: (0, 0)),
        ],
        out_specs=[
            pl.BlockSpec((block_rows, dout), lambda i: (i, 0)),
            pl.BlockSpec((block_rows, dout), lambda i: (i, 0)),
        ],
        out_shape=[out_sds, out_sds],
    )(x, wl, bl.reshape(1, -1), wr, br.reshape(1, -1))


def _sread(ref, i):
    return ref[pl.ds(i, 16)][0]


def _allsum(v, lanes):
    for s in (8, 4, 2, 1):
        v = v + v.at[lanes ^ s].get(mode="promise_in_bounds")
    return v


@functools.lru_cache(maxsize=None)
def _make_edge_kernel(D, relu):
    C = max(16, min(128, (49152 // D) // 16 * 16))
    JD = D // 16
    U = 8 if JD % 8 == 0 else (4 if JD % 4 == 0 else 1)
    mesh = plsc.VectorSubcoreMesh(core_axis_name="c", subcore_axis_name="s")

    def body(xl_h, xr_h, att_h, bias_h, src_h, dst_h, eoff_h, out_h,
             srcbuf, dstbuf, xjb, xib, attv, biasv, acc,
             orow, eoffv, sem_j, sem_i):
        lanes = lax.iota(jnp.int32, 16)
        wid = lax.axis_index("s") * 2 + lax.axis_index("c")
        pltpu.sync_copy(eoff_h, eoffv)
        pltpu.sync_copy(att_h, attv)
        pltpu.sync_copy(bias_h, biasv)
        e0 = _sread(eoffv, wid)
        e1 = _sread(eoffv, wid + 1)
        for j in range(JD):
            acc[pl.ds(16 * j, 16)] = jnp.zeros((16,), jnp.float32)
        g0 = (e0 // 8) * 8
        nchunks = jnp.where(e1 > e0, (e1 - g0 + C - 1) // C, 0)

        def fin_effects(cur_dst, den_v):
            rcp = 1.0 / den_v
            for j in range(JD):
                o_v = acc[pl.ds(16 * j, 16)] * rcp + biasv[pl.ds(16 * j, 16)]
                if relu:
                    o_v = jnp.maximum(o_v, 0.0)
                orow[0, pl.ds(16 * j, 16)] = o_v
            pltpu.sync_copy(orow, out_h.at[pl.ds(cur_dst, 1)])

        def chunk_body(c, st):
            g = pl.multiple_of(g0 + c * C, 8)
            start = jnp.maximum(e0 - g, 0)
            end = jnp.minimum(e1 - g, C)
            pltpu.sync_copy(src_h.at[pl.ds(g, C)], srcbuf)
            pltpu.sync_copy(dst_h.at[pl.ds(g, C + 16)], dstbuf)
            cj = pltpu.async_copy(xl_h.at[srcbuf], xjb, sem_j)
            ci = pltpu.async_copy(xr_h.at[dstbuf.at[pl.ds(0, C)]], xib, sem_i)
            cj.wait()
            ci.wait()

            def edge(e, s2):
                cur_dst, m, den_v = s2
                dcur = _sread(dstbuf, e)
                zero16 = jnp.zeros((16,), jnp.float32)
                NP = 4 if U >= 4 else 1

                @plsc.parallel_loop(0, JD, step=U, unroll=2,
                                    carry=tuple(zero16 for _ in range(NP)))
                def parts(b, parts):
                    parts = list(parts)
                    base = b * 16
                    for t in range(U):
                        off = base + 16 * t
                        xj_v = xjb[e, pl.ds(off, 16)]
                        xi_v = xib[e, pl.ds(off, 16)]
                        z = xi_v + xj_v
                        z = jnp.maximum(z, 0.2 * z)
                        parts[t % NP] = parts[t % NP] + z * attv[pl.ds(off, 16)]
                    return tuple(parts)

                sv = parts[0]
                for t in range(1, NP):
                    sv = sv + parts[t]
                s = _allsum(sv, lanes)[0]
                fin = (dcur != cur_dst) & (cur_dst >= 0)

                @pl.when(fin)
                def _():
                    fin_effects(cur_dst, den_v)

                fresh = dcur != cur_dst
                m = jnp.where(fresh, NEG, m)
                den_v = jnp.where(fresh, jnp.zeros((16,), jnp.float32), den_v)
                mn = jnp.maximum(m, s)
                r_v = jnp.exp(jnp.full((16,), m - mn, jnp.float32))
                w_v = jnp.exp(jnp.full((16,), s - mn, jnp.float32))
                den_v = den_v * r_v + w_v

                @pl.when(mn > m)
                def _():
                    @plsc.parallel_loop(0, JD, step=U, unroll=2)
                    def _rblk(b):
                        base = b * 16
                        for t in range(U):
                            off = base + 16 * t
                            acc[pl.ds(off, 16)] = acc[pl.ds(off, 16)] * r_v

                @plsc.parallel_loop(0, JD, step=U, unroll=2)
                def _ablk(b):
                    base = b * 16
                    for t in range(U):
                        off = base + 16 * t
                        acc[pl.ds(off, 16)] = (
                            acc[pl.ds(off, 16)]
                            + w_v * xjb[e, pl.ds(off, 16)])

                return (dcur, mn, den_v)

            return lax.fori_loop(start, end, edge, st)

        init = (jnp.int32(-1), jnp.float32(NEG),
                jnp.zeros((16,), jnp.float32))
        cur_dst, m, den_v = lax.fori_loop(0, nchunks, chunk_body, init)

        @pl.when(cur_dst >= 0)
        def _():
            fin_effects(cur_dst, den_v)

    return pl.kernel(
        body,
        mesh=mesh,
        compiler_params=pltpu.CompilerParams(use_tc_tiling_on_sc=False),
        out_type=jax.ShapeDtypeStruct((N_NODES, D), jnp.float32),
        scratch_types=[
            pltpu.VMEM((C,), jnp.int32),
            pltpu.VMEM((C + 16,), jnp.int32),
            pltpu.VMEM((C, D), jnp.float32),
            pltpu.VMEM((C, D), jnp.float32),
            pltpu.VMEM((D,), jnp.float32),
            pltpu.VMEM((D,), jnp.float32),
            pltpu.VMEM((D,), jnp.float32),
            pltpu.VMEM((1, D), jnp.float32),
            pltpu.VMEM((64,), jnp.int32),
            pltpu.SemaphoreType.DMA,
            pltpu.SemaphoreType.DMA,
        ],
    )


def _pad_cols(w, d):
    dout, din = w.shape
    wt = w.T
    if d > dout:
        wt = jnp.pad(wt, ((0, 0), (0, d - dout)))
    return wt


def _pad_vec(v, d):
    if d > v.shape[0]:
        v = jnp.pad(v, (0, d - v.shape[0]))
    return v


def _layer(x, srcs, dsts, eoff, p, relu):
    dout = p['Wl'].shape[0]
    D = max(16, -(-dout // 16) * 16)
    wl = _pad_cols(p['Wl'], D)
    wr = _pad_cols(p['Wr'], D)
    bl = _pad_vec(p['bl'], D)
    br = _pad_vec(p['br'], D)
    att = _pad_vec(p['att'], D)
    bias = _pad_vec(p['bias'], D)
    xl, xr = _linear_pair(x, wl, bl, wr, br)
    edge_fn = _make_edge_kernel(D, relu)
    out = edge_fn(xl, xr, att, bias, srcs, dsts, eoff)
    return out[:N_NODES]


def kernel(x, edge_index, params):
    n = x.shape[0]
    ei = edge_index.astype(jnp.int32)
    loop = jnp.arange(n, dtype=jnp.int32)
    src = jnp.concatenate([ei[0], loop])
    dst = jnp.concatenate([ei[1], loop])
    key = jnp.sort(dst * 16384 + src)
    dsts = key >> 14
    srcs = key & 16383
    bounds = jnp.array([w * n // NW for w in range(NW + 1)], jnp.int32)
    eoff = jnp.searchsorted(dsts, bounds).astype(jnp.int32)
    eoff = jnp.concatenate([eoff, jnp.zeros((64 - NW - 1,), jnp.int32)])
    srcs = jnp.concatenate([srcs, jnp.zeros((EPAD,), jnp.int32)])
    dsts = jnp.concatenate([dsts, jnp.zeros((EPAD,), jnp.int32)])

    out = x
    for i in range(4):
        out = _layer(out, srcs, dsts, eoff, params[i], True)
    coords = _layer(out, srcs, dsts, eoff, params[4], False)
    return (out, coords[:, :3])


# speedup vs baseline: 1.4450x; 1.0399x over previous
import functools

import jax
import jax.numpy as jnp
from jax import lax
from jax.experimental import pallas as pl
from jax.experimental.pallas import tpu as pltpu
from jax.experimental.pallas import tpu_sc as plsc

N_NODES = 10000
N_EDGES = 320000
NW = 32
EPAD = 256
NEG = -1e30


def _mm_body(x_ref, wl_ref, bl_ref, wr_ref, br_ref, ol_ref, or_ref):
    x = x_ref[...]
    ol_ref[...] = (
        jnp.dot(x, wl_ref[...], preferred_element_type=jnp.float32) + bl_ref[...]
    )
    or_ref[...] = (
        jnp.dot(x, wr_ref[...], preferred_element_type=jnp.float32) + br_ref[...]
    )


def _linear_pair(x, wl, bl, wr, br, block_rows=1000):
    n, din = x.shape
    dout = wl.shape[1]
    out_sds = jax.ShapeDtypeStruct((n, dout), jnp.float32)
    return pl.pallas_call(
        _mm_body,
        grid=(n // block_rows,),
        in_specs=[
            pl.BlockSpec((block_rows, din), lambda i: (i, 0)),
            pl.BlockSpec((din, dout), lambda i: (0, 0)),
            pl.BlockSpec((1, dout), lambda i: (0, 0)),
            pl.BlockSpec((din, dout), lambda i: (0, 0)),
            pl.BlockSpec((1, dout), lambda i: (0, 0)),
        ],
        out_specs=[
            pl.BlockSpec((block_rows, dout), lambda i: (i, 0)),
            pl.BlockSpec((block_rows, dout), lambda i: (i, 0)),
        ],
        out_shape=[out_sds, out_sds],
    )(x, wl, bl.reshape(1, -1), wr, br.reshape(1, -1))


def _sread(ref, i):
    return ref[pl.ds(i, 16)][0]


def _allsum(v, lanes):
    for s in (8, 4, 2, 1):
        v = v + v.at[lanes ^ s].get(mode="promise_in_bounds")
    return v


@functools.lru_cache(maxsize=None)
def _make_edge_kernel(D, relu):
    C = max(16, min(128, (49152 // D) // 16 * 16))
    JD = D // 16
    U = 8 if JD % 8 == 0 else (4 if JD % 4 == 0 else 1)
    mesh = plsc.VectorSubcoreMesh(core_axis_name="c", subcore_axis_name="s")

    def body(xl_h, xr_h, att_h, bias_h, src_h, dst_h, eoff_h, out_h,
             srcbuf, dstbuf, xjb, xib, attv, biasv, acc,
             orow, eoffv, sem_j, sem_i):
        lanes = lax.iota(jnp.int32, 16)
        wid = lax.axis_index("s") * 2 + lax.axis_index("c")
        pltpu.sync_copy(eoff_h, eoffv)
        pltpu.sync_copy(att_h, attv)
        pltpu.sync_copy(bias_h, biasv)
        e0 = _sread(eoffv, wid)
        e1 = _sread(eoffv, wid + 1)
        for j in range(JD):
            acc[pl.ds(16 * j, 16)] = jnp.zeros((16,), jnp.float32)
        g0 = (e0 // 8) * 8
        nchunks = jnp.where(e1 > e0, (e1 - g0 + C - 1) // C, 0)

        def fin_effects(cur_dst, den_v):
            rcp = 1.0 / den_v
            for j in range(JD):
                o_v = acc[pl.ds(16 * j, 16)] * rcp + biasv[pl.ds(16 * j, 16)]
                if relu:
                    o_v = jnp.maximum(o_v, 0.0)
                orow[0, pl.ds(16 * j, 16)] = o_v
            pltpu.sync_copy(orow, out_h.at[pl.ds(cur_dst, 1)])

        def chunk_body(c, st):
            g = pl.multiple_of(g0 + c * C, 8)
            start = jnp.maximum(e0 - g, 0)
            end = jnp.minimum(e1 - g, C)
            pltpu.sync_copy(src_h.at[pl.ds(g, C)], srcbuf)
            pltpu.sync_copy(dst_h.at[pl.ds(g, C + 16)], dstbuf)
            cj = pltpu.async_copy(xl_h.at[srcbuf], xjb, sem_j)
            ci = pltpu.async_copy(xr_h.at[dstbuf.at[pl.ds(0, C)]], xib, sem_i)
            cj.wait()
            ci.wait()

            def edge(e, s2):
                cur_dst, m, den_v = s2
                dcur = _sread(dstbuf, e)
                zero16 = jnp.zeros((16,), jnp.float32)
                NP = 4 if U >= 4 else 1

                @plsc.parallel_loop(0, JD, step=U, unroll=4,
                                    carry=tuple(zero16 for _ in range(NP)))
                def parts(b, parts):
                    parts = list(parts)
                    base = b * 16
                    for t in range(U):
                        off = base + 16 * t
                        xj_v = xjb[e, pl.ds(off, 16)]
                        xi_v = xib[e, pl.ds(off, 16)]
                        z = xi_v + xj_v
                        z = jnp.maximum(z, 0.2 * z)
                        parts[t % NP] = parts[t % NP] + z * attv[pl.ds(off, 16)]
                    return tuple(parts)

                sv = parts[0]
                for t in range(1, NP):
                    sv = sv + parts[t]
                s = _allsum(sv, lanes)[0]
                fin = (dcur != cur_dst) & (cur_dst >= 0)

                @pl.when(fin)
                def _():
                    fin_effects(cur_dst, den_v)

                fresh = dcur != cur_dst
                m = jnp.where(fresh, NEG, m)
                den_v = jnp.where(fresh, jnp.zeros((16,), jnp.float32), den_v)
                mn = jnp.maximum(m, s)
                r_v = jnp.exp(jnp.full((16,), m - mn, jnp.float32))
                w_v = jnp.exp(jnp.full((16,), s - mn, jnp.float32))
                den_v = den_v * r_v + w_v

                @pl.when(mn > m)
                def _():
                    @plsc.parallel_loop(0, JD, step=U, unroll=4)
                    def _rblk(b):
                        base = b * 16
                        for t in range(U):
                            off = base + 16 * t
                            acc[pl.ds(off, 16)] = acc[pl.ds(off, 16)] * r_v

                @plsc.parallel_loop(0, JD, step=U, unroll=4)
                def _ablk(b):
                    base = b * 16
                    for t in range(U):
                        off = base + 16 * t
                        acc[pl.ds(off, 16)] = (
                            acc[pl.ds(off, 16)]
                            + w_v * xjb[e, pl.ds(off, 16)])

                return (dcur, mn, den_v)

            return lax.fori_loop(start, end, edge, st)

        init = (jnp.int32(-1), jnp.float32(NEG),
                jnp.zeros((16,), jnp.float32))
        cur_dst, m, den_v = lax.fori_loop(0, nchunks, chunk_body, init)

        @pl.when(cur_dst >= 0)
        def _():
            fin_effects(cur_dst, den_v)

    return pl.kernel(
        body,
        mesh=mesh,
        compiler_params=pltpu.CompilerParams(use_tc_tiling_on_sc=False),
        out_type=jax.ShapeDtypeStruct((N_NODES, D), jnp.float32),
        scratch_types=[
            pltpu.VMEM((C,), jnp.int32),
            pltpu.VMEM((C + 16,), jnp.int32),
            pltpu.VMEM((C, D), jnp.float32),
            pltpu.VMEM((C, D), jnp.float32),
            pltpu.VMEM((D,), jnp.float32),
            pltpu.VMEM((D,), jnp.float32),
            pltpu.VMEM((D,), jnp.float32),
            pltpu.VMEM((1, D), jnp.float32),
            pltpu.VMEM((64,), jnp.int32),
            pltpu.SemaphoreType.DMA,
            pltpu.SemaphoreType.DMA,
        ],
    )


def _pad_cols(w, d):
    dout, din = w.shape
    wt = w.T
    if d > dout:
        wt = jnp.pad(wt, ((0, 0), (0, d - dout)))
    return wt


def _pad_vec(v, d):
    if d > v.shape[0]:
        v = jnp.pad(v, (0, d - v.shape[0]))
    return v


def _layer(x, srcs, dsts, eoff, p, relu):
    dout = p['Wl'].shape[0]
    D = max(16, -(-dout // 16) * 16)
    wl = _pad_cols(p['Wl'], D)
    wr = _pad_cols(p['Wr'], D)
    bl = _pad_vec(p['bl'], D)
    br = _pad_vec(p['br'], D)
    att = _pad_vec(p['att'], D)
    bias = _pad_vec(p['bias'], D)
    xl, xr = _linear_pair(x, wl, bl, wr, br)
    edge_fn = _make_edge_kernel(D, relu)
    out = edge_fn(xl, xr, att, bias, srcs, dsts, eoff)
    return out[:N_NODES]


def kernel(x, edge_index, params):
    n = x.shape[0]
    ei = edge_index.astype(jnp.int32)
    loop = jnp.arange(n, dtype=jnp.int32)
    src = jnp.concatenate([ei[0], loop])
    dst = jnp.concatenate([ei[1], loop])
    key = jnp.sort(dst * 16384 + src)
    dsts = key >> 14
    srcs = key & 16383
    bounds = jnp.array([w * n // NW for w in range(NW + 1)], jnp.int32)
    eoff = jnp.searchsorted(dsts, bounds).astype(jnp.int32)
    eoff = jnp.concatenate([eoff, jnp.zeros((64 - NW - 1,), jnp.int32)])
    srcs = jnp.concatenate([srcs, jnp.zeros((EPAD,), jnp.int32)])
    dsts = jnp.concatenate([dsts, jnp.zeros((EPAD,), jnp.int32)])

    out = x
    for i in range(4):
        out = _layer(out, srcs, dsts, eoff, params[i], True)
    coords = _layer(out, srcs, dsts, eoff, params[4], False)
    return (out, coords[:, :3])
